# Initial kernel scaffold; baseline (speedup 1.0000x reference)
import jax
import jax.numpy as jnp
from jax.experimental import pallas as pl


def kernel(graph, table):
    raise NotImplementedError("write your pallas kernel here")


# speedup vs baseline: 1.9953x; 1.9953x over previous
import functools

import jax
import jax.numpy as jnp
from jax import lax
from jax.experimental import pallas as pl
from jax.experimental.pallas import tpu as pltpu
from jax.experimental.pallas import tpu_sc as plsc

VOCAB = 1000000
EMBED = 32
E = 800000
TOTAL = 2 * E
NW = 32
RW = TOTAL // NW
CHUNK = 80
NCHUNK = RW // CHUNK
assert NCHUNK * CHUNK == RW


def _make_kernel():
    mesh = plsc.VectorSubcoreMesh(core_axis_name="c", subcore_axis_name="s")

    @functools.partial(
        pl.kernel,
        mesh=mesh,
        compiler_params=pltpu.CompilerParams(use_tc_tiling_on_sc=False),
        out_type=jax.ShapeDtypeStruct((TOTAL, EMBED), jnp.float32),
        scratch_types=[
            pltpu.VMEM((CHUNK,), jnp.int32),
            pltpu.VMEM((CHUNK, EMBED), jnp.float32),
            pltpu.SemaphoreType.DMA,
        ],
    )
    def gather_kernel(idx_hbm, table_hbm, out_hbm, idx_v, rows_v, sem):
        wid = lax.axis_index("s") * 2 + lax.axis_index("c")
        base = wid * RW

        def chunk_body(i, carry):
            off = base + i * CHUNK
            pltpu.sync_copy(idx_hbm.at[pl.ds(off, CHUNK)], idx_v)
            pltpu.async_copy(table_hbm.at[idx_v], rows_v, sem).wait()
            pltpu.sync_copy(rows_v, out_hbm.at[pl.ds(off, CHUNK)])
            return carry

        lax.fori_loop(0, NCHUNK, chunk_body, 0)

    return gather_kernel


_gather = _make_kernel()


def kernel(graph, table):
    idx = graph.reshape(-1).astype(jnp.int32)
    out = _gather(idx, table)
    return out.reshape(E, 2 * EMBED)


# speedup vs baseline: 2.5851x; 1.2956x over previous
import functools

import jax
import jax.numpy as jnp
from jax import lax
from jax.experimental import pallas as pl
from jax.experimental.pallas import tpu as pltpu
from jax.experimental.pallas import tpu_sc as plsc

VOCAB = 1000000
EMBED = 32
E = 800000
TOTAL = 2 * E
NW = 32
RW = TOTAL // NW
CHUNK = 80
NCHUNK = RW // CHUNK
NBUF = 5
NGROUP = NCHUNK // NBUF
assert NCHUNK * CHUNK == RW and NGROUP * NBUF == NCHUNK


def _make_kernel():
    mesh = plsc.VectorSubcoreMesh(core_axis_name="c", subcore_axis_name="s")

    @functools.partial(
        pl.kernel,
        mesh=mesh,
        compiler_params=pltpu.CompilerParams(use_tc_tiling_on_sc=False),
        out_type=jax.ShapeDtypeStruct((TOTAL, EMBED), jnp.float32),
        scratch_types=[
            pltpu.VMEM((2, NBUF, CHUNK), jnp.int32),
            pltpu.VMEM((2, NBUF, CHUNK, EMBED), jnp.float32),
            pltpu.SemaphoreType.DMA,
            pltpu.SemaphoreType.DMA,
            pltpu.SemaphoreType.DMA,
        ],
    )
    def gather_kernel(idx_hbm, table_hbm, out_hbm, idx_v, rows_v, sem_i,
                      sem_g, sem_o):
        wid = lax.axis_index("s") * 2 + lax.axis_index("c")
        chunk0 = wid * NCHUNK

        def fire_idx(g, p):
            pltpu.async_copy(
                idx_hbm.at[pl.ds(chunk0 + g * NBUF, NBUF)], idx_v.at[p],
                sem_i)

        def wait_idx(p):
            pltpu.make_async_copy(
                idx_hbm.at[pl.ds(chunk0, NBUF)], idx_v.at[p], sem_i).wait()

        def fire_gathers(p):
            for b in range(NBUF):
                pltpu.async_copy(
                    table_hbm.at[idx_v.at[p, b]], rows_v.at[p, b], sem_g)

        def wait_gathers(p):
            for b in range(NBUF):
                pltpu.make_async_copy(
                    table_hbm.at[idx_v.at[p, b]], rows_v.at[p, b],
                    sem_g).wait()

        def fire_stores(g, p):
            base = (chunk0 + g * NBUF) * CHUNK
            for b in range(NBUF):
                pltpu.async_copy(
                    rows_v.at[p, b],
                    out_hbm.at[pl.ds(base + b * CHUNK, CHUNK)], sem_o)

        def wait_stores(g, p):
            base = (chunk0 + g * NBUF) * CHUNK
            for b in range(NBUF):
                pltpu.make_async_copy(
                    rows_v.at[p, b],
                    out_hbm.at[pl.ds(base + b * CHUNK, CHUNK)], sem_o).wait()

        fire_idx(0, 0)
        wait_idx(0)
        fire_gathers(0)
        fire_idx(1, 1)

        def body(g, carry):
            p = g % 2
            q = 1 - p
            wait_gathers(p)
            pl.when(g > 0)(lambda: wait_stores(g - 1, q))
            fire_stores(g, p)

            def start_next():
                wait_idx(q)
                fire_gathers(q)

            pl.when(g < NGROUP - 1)(start_next)
            pl.when(g < NGROUP - 2)(lambda: fire_idx(g + 2, p))
            return carry

        lax.fori_loop(0, NGROUP, body, 0)
        wait_stores(NGROUP - 1, (NGROUP - 1) % 2)

    return gather_kernel


_gather = _make_kernel()


def kernel(graph, table):
    idx = graph.reshape(-1, CHUNK).astype(jnp.int32)
    out = _gather(idx, table)
    return out.reshape(E, 2 * EMBED)


# speedup vs baseline: 2.6503x; 1.0252x over previous
import functools

import jax
import jax.numpy as jnp
from jax import lax
from jax.experimental import pallas as pl
from jax.experimental.pallas import tpu as pltpu
from jax.experimental.pallas import tpu_sc as plsc

VOCAB = 1000000
EMBED = 32
E = 800000
TOTAL = 2 * E
NW = 32
RW = TOTAL // NW
CHUNK = 80
NCHUNK = RW // CHUNK
NBUF = 5
NGROUP = NCHUNK // NBUF
DEPTH = 4
assert NCHUNK * CHUNK == RW and NGROUP * NBUF == NCHUNK


def _make_kernel():
    mesh = plsc.VectorSubcoreMesh(core_axis_name="c", subcore_axis_name="s")

    @functools.partial(
        pl.kernel,
        mesh=mesh,
        compiler_params=pltpu.CompilerParams(use_tc_tiling_on_sc=False),
        out_type=jax.ShapeDtypeStruct((TOTAL, EMBED), jnp.float32),
        scratch_types=[
            pltpu.VMEM((DEPTH, NBUF, CHUNK), jnp.int32),
            pltpu.VMEM((DEPTH, NBUF, CHUNK, EMBED), jnp.float32),
            pltpu.SemaphoreType.DMA,
            pltpu.SemaphoreType.DMA((DEPTH,)),
            pltpu.SemaphoreType.DMA((DEPTH,)),
        ],
    )
    def gather_kernel(idx_hbm, table_hbm, out_hbm, idx_v, rows_v, sem_i,
                      sem_g, sem_o):
        wid = lax.axis_index("s") * 2 + lax.axis_index("c")
        chunk0 = wid * NCHUNK

        def fire_idx(g):
            pltpu.async_copy(
                idx_hbm.at[pl.ds(chunk0 + g * NBUF, NBUF)],
                idx_v.at[g % DEPTH], sem_i)

        def wait_idx(g):
            pltpu.make_async_copy(
                idx_hbm.at[pl.ds(chunk0, NBUF)], idx_v.at[g % DEPTH],
                sem_i).wait()

        def fire_gathers(g):
            p = g % DEPTH
            for b in range(NBUF):
                pltpu.async_copy(
                    table_hbm.at[idx_v.at[p, b]], rows_v.at[p, b],
                    sem_g.at[p])

        def wait_gathers(g):
            p = g % DEPTH
            for b in range(NBUF):
                pltpu.make_async_copy(
                    table_hbm.at[idx_v.at[p, b]], rows_v.at[p, b],
                    sem_g.at[p]).wait()

        def fire_stores(g):
            p = g % DEPTH
            base = (chunk0 + g * NBUF) * CHUNK
            for b in range(NBUF):
                pltpu.async_copy(
                    rows_v.at[p, b],
                    out_hbm.at[pl.ds(base + b * CHUNK, CHUNK)], sem_o.at[p])

        def wait_stores(g):
            p = g % DEPTH
            base = (chunk0 + g * NBUF) * CHUNK
            for b in range(NBUF):
                pltpu.make_async_copy(
                    rows_v.at[p, b],
                    out_hbm.at[pl.ds(base + b * CHUNK, CHUNK)],
                    sem_o.at[p]).wait()

        fire_idx(0)
        for h in range(1, DEPTH - 1):
            wait_idx(h - 1)
            fire_gathers(h - 1)
            fire_idx(h)
        wait_idx(DEPTH - 2)
        fire_gathers(DEPTH - 2)
        fire_idx(DEPTH - 1)

        def body(g, carry):
            wait_gathers(g)
            pl.when(g > 0)(lambda: wait_stores(g - 1))
            fire_stores(g)

            def start_next():
                wait_idx(g + DEPTH - 1)
                fire_gathers(g + DEPTH - 1)

            pl.when(g + DEPTH - 1 < NGROUP)(start_next)
            pl.when(g + DEPTH < NGROUP)(lambda: fire_idx(g + DEPTH))
            return carry

        lax.fori_loop(0, NGROUP, body, 0)
        wait_stores(NGROUP - 1)

    return gather_kernel


_gather = _make_kernel()


def kernel(graph, table):
    idx = graph.reshape(-1, CHUNK).astype(jnp.int32)
    out = _gather(idx, table)
    return out.reshape(E, 2 * EMBED)


# speedup vs baseline: 3.0233x; 1.1407x over previous
import functools

import jax
import jax.numpy as jnp
from jax import lax
from jax.experimental import pallas as pl
from jax.experimental.pallas import tpu as pltpu
from jax.experimental.pallas import tpu_sc as plsc

VOCAB = 1000000
EMBED = 32
E = 800000
NB = E // 128
NR = 2 * NB
CHUNK = 128
DEPTH = 4
NCH_HI = 391
NCH_LO = 390
L = 16


def _make_kernel():
    mesh = plsc.VectorSubcoreMesh(core_axis_name="c", subcore_axis_name="s")

    @functools.partial(
        pl.kernel,
        mesh=mesh,
        compiler_params=pltpu.CompilerParams(use_tc_tiling_on_sc=False,
                                             needs_layout_passes=False),
        out_type=jax.ShapeDtypeStruct((8, NB, 8, 128), jnp.float32),
        scratch_types=[
            pltpu.VMEM((DEPTH, CHUNK), jnp.int32),
            pltpu.VMEM((DEPTH * CHUNK, EMBED), jnp.float32),
            pltpu.VMEM((4, 8, 128), jnp.float32),
            pltpu.SemaphoreType.DMA,
            pltpu.SemaphoreType.DMA((DEPTH,)),
            pltpu.SemaphoreType.DMA,
        ],
    )
    def gather_kernel(idx_hbm, table_hbm, out_hbm, idx_v, rows_v, tiles_v,
                      sem_i, sem_g, sem_o):
        wid = lax.axis_index("s") * 2 + lax.axis_index("c")
        nch = lax.select(wid < 20, NCH_HI, NCH_LO)
        r0 = lax.select(wid < 20, wid * NCH_HI,
                        20 * NCH_HI + (wid - 20) * NCH_LO)
        lane = lax.broadcasted_iota(jnp.int32, (L,), 0)

        def fire_idx(t):
            pltpu.async_copy(idx_hbm.at[r0 + t], idx_v.at[t % DEPTH], sem_i)

        def wait_idx(t):
            pltpu.make_async_copy(idx_hbm.at[r0], idx_v.at[t % DEPTH],
                                  sem_i).wait()

        def fire_gather(t):
            p = t % DEPTH
            pltpu.async_copy(
                table_hbm.at[idx_v.at[p]],
                rows_v.at[pl.ds(p * CHUNK, CHUNK)], sem_g.at[p])

        def wait_gather(t):
            p = t % DEPTH
            pltpu.make_async_copy(
                table_hbm.at[idx_v.at[p]],
                rows_v.at[pl.ds(p * CHUNK, CHUNK)], sem_g.at[p]).wait()

        def fire_stores(t):
            r = r0 + t
            b = r // 2
            jt0 = 4 * (r % 2)
            for jt in range(4):
                pltpu.async_copy(tiles_v.at[jt], out_hbm.at[jt0 + jt, b],
                                 sem_o)

        def wait_stores(t):
            r = r0 + t
            b = r // 2
            jt0 = 4 * (r % 2)
            for jt in range(4):
                pltpu.make_async_copy(tiles_v.at[jt],
                                      out_hbm.at[jt0 + jt, b], sem_o).wait()

        def transpose(t):
            row0 = (t % DEPTH) * CHUNK
            for c in range(EMBED):
                cvec = jnp.full((L,), c, jnp.int32)
                for k in range(CHUNK // L):
                    rvec = lane + (row0 + k * L)
                    vals = plsc.load_gather(rows_v, [rvec, cvec])
                    tiles_v[c // 8, c % 8, pl.ds(k * L, L)] = vals

        for h in range(DEPTH - 1):
            fire_idx(h)
            wait_idx(h)
            fire_gather(h)
        fire_idx(DEPTH - 1)

        def body(t, carry):
            wait_gather(t)
            pl.when(t >= 1)(lambda: wait_stores(t - 1))
            transpose(t)
            fire_stores(t)

            def start_next():
                wait_idx(t + DEPTH - 1)
                fire_gather(t + DEPTH - 1)

            pl.when(t + DEPTH - 1 < nch)(start_next)
            pl.when(t + DEPTH < nch)(lambda: fire_idx(t + DEPTH))
            return carry

        lax.fori_loop(0, nch, body, 0)
        wait_stores(nch - 1)

    return gather_kernel


_gather = _make_kernel()


def kernel(graph, table):
    idx = graph.reshape(NB, 128, 2).transpose(0, 2, 1).reshape(NR, 128)
    out_p = _gather(idx.astype(jnp.int32), table)
    return out_p.transpose(1, 3, 0, 2).reshape(E, 2 * EMBED)


# speedup vs baseline: 5.7258x; 1.8939x over previous
import functools

import jax
import jax.numpy as jnp
from jax import lax
from jax.experimental import pallas as pl
from jax.experimental.pallas import tpu as pltpu
from jax.experimental.pallas import tpu_sc as plsc

VOCAB = 1000000
EMBED = 32
E = 800000
NB = E // 128
NR = 2 * NB
CHUNK = 128
DEPTH = 4
NCH_HI = 391
NCH_LO = 390
L = 16


def _make_kernel():
    mesh = plsc.VectorSubcoreMesh(core_axis_name="c", subcore_axis_name="s")

    @functools.partial(
        pl.kernel,
        mesh=mesh,
        compiler_params=pltpu.CompilerParams(use_tc_tiling_on_sc=False,
                                             needs_layout_passes=False),
        out_type=jax.ShapeDtypeStruct((8, NB, 8, 128), jnp.float32),
        scratch_types=[
            pltpu.VMEM((DEPTH, CHUNK), jnp.int32),
            pltpu.VMEM((DEPTH * CHUNK, EMBED), jnp.float32),
            pltpu.VMEM((2 * EMBED, 129), jnp.float32),
            pltpu.SemaphoreType.DMA,
            pltpu.SemaphoreType.DMA((DEPTH,)),
            pltpu.SemaphoreType.DMA,
        ],
    )
    def gather_kernel(idx_hbm, table_hbm, out_hbm, idx_v, rows_v, tiles_v,
                      sem_i, sem_g, sem_o):
        wid = lax.axis_index("s") * 2 + lax.axis_index("c")
        nch = lax.select(wid < 20, NCH_HI, NCH_LO)
        r0 = lax.select(wid < 20, wid * NCH_HI,
                        20 * NCH_HI + (wid - 20) * NCH_LO)
        lane = lax.broadcasted_iota(jnp.int32, (L,), 0)

        def fire_idx(t):
            pltpu.async_copy(idx_hbm.at[r0 + t], idx_v.at[t % DEPTH], sem_i)

        def wait_idx(t):
            pltpu.make_async_copy(idx_hbm.at[r0], idx_v.at[t % DEPTH],
                                  sem_i).wait()

        def fire_gather(t):
            p = t % DEPTH
            pltpu.async_copy(
                table_hbm.at[idx_v.at[p]],
                rows_v.at[pl.ds(p * CHUNK, CHUNK)], sem_g.at[p])

        def wait_gather(t):
            p = t % DEPTH
            pltpu.make_async_copy(
                table_hbm.at[idx_v.at[p]],
                rows_v.at[pl.ds(p * CHUNK, CHUNK)], sem_g.at[p]).wait()

        def fire_stores(t):
            r = r0 + t
            b = r // 2
            jt0 = 4 * (r % 2)
            for jt in range(4):
                pltpu.async_copy(
                    tiles_v.at[pl.ds(jt * 8, 8), pl.ds(0, 128)],
                    out_hbm.at[jt0 + jt, b], sem_o)

        def wait_stores(t):
            r = r0 + t
            b = r // 2
            jt0 = 4 * (r % 2)
            for jt in range(4):
                pltpu.make_async_copy(
                    tiles_v.at[pl.ds(jt * 8, 8), pl.ds(0, 128)],
                    out_hbm.at[jt0 + jt, b], sem_o).wait()

        def transpose(t):
            row0 = (t % DEPTH) * CHUNK
            rvecs = [lane + 16 * k for k in range(EMBED // L)]
            cvecs = [lane + 16 * k for k in range(EMBED // L)]
            for i in range(CHUNK):
                ivec = jnp.full((L,), i, jnp.int32)
                svec = jnp.full((L,), row0 + i, jnp.int32)
                for k in range(EMBED // L):
                    vals = plsc.load_gather(rows_v, [svec, cvecs[k]])
                    plsc.store_scatter(tiles_v, [rvecs[k], ivec], vals)

        for h in range(DEPTH - 1):
            fire_idx(h)
            wait_idx(h)
            fire_gather(h)
        fire_idx(DEPTH - 1)

        def body(t, carry):
            wait_gather(t)
            pl.when(t >= 1)(lambda: wait_stores(t - 1))
            transpose(t)
            fire_stores(t)

            def start_next():
                wait_idx(t + DEPTH - 1)
                fire_gather(t + DEPTH - 1)

            pl.when(t + DEPTH - 1 < nch)(start_next)
            pl.when(t + DEPTH < nch)(lambda: fire_idx(t + DEPTH))
            return carry

        lax.fori_loop(0, nch, body, 0)
        wait_stores(nch - 1)

    return gather_kernel


_gather = _make_kernel()


def kernel(graph, table):
    idx = graph.reshape(NB, 128, 2).transpose(0, 2, 1).reshape(NR, 128)
    out_p = _gather(idx.astype(jnp.int32), table)
    return out_p.transpose(1, 3, 0, 2).reshape(E, 2 * EMBED)


# speedup vs baseline: 5.8456x; 1.0209x over previous
import functools

import jax
import jax.numpy as jnp
from jax import lax
from jax.experimental import pallas as pl
from jax.experimental.pallas import tpu as pltpu
from jax.experimental.pallas import tpu_sc as plsc

VOCAB = 1000000
EMBED = 32
E = 800000
NB = E // 128
NR = 2 * NB
CHUNK = 128
DEPTH = 4
NCH_HI = 391
NCH_LO = 390
L = 16


def _make_kernel():
    mesh = plsc.VectorSubcoreMesh(core_axis_name="c", subcore_axis_name="s")

    @functools.partial(
        pl.kernel,
        mesh=mesh,
        compiler_params=pltpu.CompilerParams(use_tc_tiling_on_sc=False,
                                             needs_layout_passes=False),
        out_type=jax.ShapeDtypeStruct((8, NB, 8, 128), jnp.float32),
        scratch_types=[
            pltpu.VMEM((DEPTH, CHUNK), jnp.int32),
            pltpu.VMEM((DEPTH * CHUNK, EMBED), jnp.float32),
            pltpu.VMEM((2 * EMBED, 129), jnp.float32),
            pltpu.SemaphoreType.DMA,
            pltpu.SemaphoreType.DMA((DEPTH,)),
            pltpu.SemaphoreType.DMA,
        ],
    )
    def gather_kernel(idx_hbm, table_hbm, out_hbm, idx_v, rows_v, tiles_v,
                      sem_i, sem_g, sem_o):
        wid = lax.axis_index("s") * 2 + lax.axis_index("c")
        nch = lax.select(wid < 20, NCH_HI, NCH_LO)
        r0 = lax.select(wid < 20, wid * NCH_HI,
                        20 * NCH_HI + (wid - 20) * NCH_LO)
        lane = lax.broadcasted_iota(jnp.int32, (L,), 0)

        def fire_idx(t):
            r = r0 + t
            pltpu.async_copy(idx_hbm.at[r // 2, r % 2], idx_v.at[t % DEPTH],
                             sem_i)

        def wait_idx(t):
            pltpu.make_async_copy(idx_hbm.at[0, 0], idx_v.at[t % DEPTH],
                                  sem_i).wait()

        def fire_gather(t):
            p = t % DEPTH
            pltpu.async_copy(
                table_hbm.at[idx_v.at[p]],
                rows_v.at[pl.ds(p * CHUNK, CHUNK)], sem_g.at[p])

        def wait_gather(t):
            p = t % DEPTH
            pltpu.make_async_copy(
                table_hbm.at[idx_v.at[p]],
                rows_v.at[pl.ds(p * CHUNK, CHUNK)], sem_g.at[p]).wait()

        def fire_stores(t):
            r = r0 + t
            b = r // 2
            jt0 = 4 * (r % 2)
            for jt in range(4):
                pltpu.async_copy(
                    tiles_v.at[pl.ds(jt * 8, 8), pl.ds(0, 128)],
                    out_hbm.at[jt0 + jt, b], sem_o)

        def wait_stores(t):
            r = r0 + t
            b = r // 2
            jt0 = 4 * (r % 2)
            for jt in range(4):
                pltpu.make_async_copy(
                    tiles_v.at[pl.ds(jt * 8, 8), pl.ds(0, 128)],
                    out_hbm.at[jt0 + jt, b], sem_o).wait()

        def transpose(t):
            row0 = (t % DEPTH) * CHUNK
            rvecs = [lane + 16 * k for k in range(EMBED // L)]
            cvecs = [lane + 16 * k for k in range(EMBED // L)]
            for i in range(CHUNK):
                ivec = jnp.full((L,), i, jnp.int32)
                svec = jnp.full((L,), row0 + i, jnp.int32)
                for k in range(EMBED // L):
                    vals = plsc.load_gather(rows_v, [svec, cvecs[k]])
                    plsc.store_scatter(tiles_v, [rvecs[k], ivec], vals)

        for h in range(DEPTH - 1):
            fire_idx(h)
            wait_idx(h)
            fire_gather(h)
        fire_idx(DEPTH - 1)

        def body(t, carry):
            wait_gather(t)
            pl.when(t >= 1)(lambda: wait_stores(t - 1))
            transpose(t)
            fire_stores(t)

            def start_next():
                wait_idx(t + DEPTH - 1)
                fire_gather(t + DEPTH - 1)

            pl.when(t + DEPTH - 1 < nch)(start_next)
            pl.when(t + DEPTH < nch)(lambda: fire_idx(t + DEPTH))
            return carry

        lax.fori_loop(0, nch, body, 0)
        wait_stores(nch - 1)

    return gather_kernel


_gather = _make_kernel()


def kernel(graph, table):
    idx = graph.reshape(NB, 128, 2).transpose(0, 2, 1)
    out_p = _gather(idx.astype(jnp.int32), table)
    return out_p.transpose(1, 3, 0, 2).reshape(E, 2 * EMBED)
